# Initial kernel scaffold; baseline (speedup 1.0000x reference)
import jax
import jax.numpy as jnp
from jax import lax
from jax.experimental import pallas as pl
from jax.experimental.pallas import tpu as pltpu, tpu_sc as plsc

BATCH = 16384
MAX_LEN = 50
EMBED_DIM = 32
NC, NS, L = 2, 16, 16
NW = NC * NS
ROWS_W = BATCH // NW
CHUNK = 16
NCHUNK = ROWS_W // CHUNK
SUBW = 100
SUB_PER_CHUNK = CHUNK * MAX_LEN // SUBW


def _sbpr_kernel(seq_hbm, next_hbm, table_hbm, out_hbm,
                 idx_v, next_idx_v, rows_v, next_rows_v, prod_v, out_v,
                 sem, nsem):
    wid = lax.axis_index("s") * NC + lax.axis_index("c")
    base = wid * ROWS_W

    pltpu.sync_copy(seq_hbm.at[pl.ds(wid * (ROWS_W * MAX_LEN // SUBW),
                                     ROWS_W * MAX_LEN // SUBW)], idx_v)
    pltpu.sync_copy(next_hbm.at[pl.ds(wid * (ROWS_W // L), ROWS_W // L)],
                    next_idx_v)

    iota = lax.iota(jnp.int32, L)
    row_half = lax.shift_right_logical(iota, 1)
    col_half = lax.mul(lax.and_(iota, 1), MAX_LEN)
    d16 = lax.mul(iota, L)

    def chunk_body(c, _):
        sub0 = c * SUB_PER_CHUNK
        for k in range(SUB_PER_CHUNK):
            pltpu.async_copy(table_hbm.at[idx_v.at[sub0 + k]],
                             rows_v.at[pl.ds(k * SUBW, SUBW)], sem)
        pltpu.async_copy(table_hbm.at[next_idx_v.at[c]], next_rows_v, nsem)
        for k in range(SUB_PER_CHUNK):
            pltpu.make_async_copy(table_hbm.at[idx_v.at[sub0 + k]],
                                  rows_v.at[pl.ds(k * SUBW, SUBW)], sem).wait()
        pltpu.make_async_copy(table_hbm.at[next_idx_v.at[c]], next_rows_v,
                              nsem).wait()

        row_vec = lax.add(row_half, sub0)
        cnt = jnp.zeros((L,), jnp.float32)
        one = jnp.ones((L,), jnp.float32)
        zero = jnp.zeros((L,), jnp.float32)
        for j in range(MAX_LEN):
            v = plsc.load_gather(idx_v, [row_vec, lax.add(col_half, j)])
            cnt = lax.add(cnt, lax.select(lax.ne(v, 0), one, zero))

        def row_body(r, _):
            b = r * MAX_LEN
            a0 = rows_v[b, pl.ds(0, L)]
            a1 = rows_v[b, pl.ds(L, L)]
            for j in range(1, MAX_LEN):
                a0 = lax.add(a0, rows_v[b + j, pl.ds(0, L)])
                a1 = lax.add(a1, rows_v[b + j, pl.ds(L, L)])
            p0 = lax.mul(a0, next_rows_v[r, pl.ds(0, L)])
            p1 = lax.mul(a1, next_rows_v[r, pl.ds(L, L)])
            plsc.store_scatter(prod_v, [lax.add(d16, r)], p0)
            plsc.store_scatter(prod_v, [lax.add(d16, r + L * L)], p1)
            return 0

        lax.fori_loop(0, CHUNK, row_body, 0)

        score = prod_v[pl.ds(0, L)]
        for d in range(1, EMBED_DIM):
            score = lax.add(score, prod_v[pl.ds(d * L, L)])
        out_v[pl.ds(c * CHUNK, CHUNK)] = lax.div(score, cnt)
        return 0

    lax.fori_loop(0, NCHUNK, chunk_body, 0)
    pltpu.sync_copy(out_v, out_hbm.at[pl.ds(base, ROWS_W)])


def kernel(item_seq, next_item, item_embed):
    seq2d = item_seq.reshape(BATCH * MAX_LEN // SUBW, SUBW)
    next2d = next_item.reshape(BATCH // L, L)
    mesh = plsc.VectorSubcoreMesh(core_axis_name="c", subcore_axis_name="s")
    f = pl.kernel(
        _sbpr_kernel,
        out_type=jax.ShapeDtypeStruct((BATCH,), jnp.float32),
        mesh=mesh,
        scratch_types=[
            pltpu.VMEM((ROWS_W * MAX_LEN // SUBW, SUBW), jnp.int32),
            pltpu.VMEM((ROWS_W // L, L), jnp.int32),
            pltpu.VMEM((CHUNK * MAX_LEN, EMBED_DIM), jnp.float32),
            pltpu.VMEM((CHUNK, EMBED_DIM), jnp.float32),
            pltpu.VMEM((CHUNK * EMBED_DIM,), jnp.float32),
            pltpu.VMEM((ROWS_W,), jnp.float32),
            pltpu.SemaphoreType.DMA,
            pltpu.SemaphoreType.DMA,
        ],
    )
    return f(seq2d, next2d, item_embed)


# speedup vs baseline: 2.7940x; 2.7940x over previous
import jax
import jax.numpy as jnp
from jax import lax
from jax.experimental import pallas as pl
from jax.experimental.pallas import tpu as pltpu, tpu_sc as plsc

BATCH = 16384
MAX_LEN = 50
EMBED_DIM = 32
NC, NS, L = 2, 16, 16
NW = NC * NS
ROWS_W = BATCH // NW
CHUNK = 16
NCHUNK = ROWS_W // CHUNK
IDX_W = ROWS_W * MAX_LEN
IDX_C = CHUNK * MAX_LEN
SUBW = 80
SUB_PER_CHUNK = IDX_C // SUBW


def _sbpr_kernel(seq_hbm, next_hbm, table_hbm, out_hbm,
                 idx_v, next_idx_v, rows_v, next_rows_v, prod_v, out_v,
                 sem, nsem):
    wid = lax.axis_index("s") * NC + lax.axis_index("c")

    pltpu.sync_copy(seq_hbm.at[pl.ds(wid * IDX_W, IDX_W)], idx_v)
    pltpu.sync_copy(next_hbm.at[pl.ds(wid * ROWS_W, ROWS_W)], next_idx_v)

    iota = lax.iota(jnp.int32, L)
    lane50 = lax.mul(iota, MAX_LEN)
    d16 = lax.mul(iota, L)

    def chunk_body(c, _):
        for k in range(SUB_PER_CHUNK):
            pltpu.async_copy(
                table_hbm.at[idx_v.at[pl.ds(c * IDX_C + k * SUBW, SUBW)]],
                rows_v.at[pl.ds(k * SUBW, SUBW)], sem)
        pltpu.async_copy(table_hbm.at[next_idx_v.at[pl.ds(c * CHUNK, CHUNK)]],
                         next_rows_v, nsem)
        for k in range(SUB_PER_CHUNK):
            pltpu.make_async_copy(
                table_hbm.at[idx_v.at[pl.ds(c * IDX_C + k * SUBW, SUBW)]],
                rows_v.at[pl.ds(k * SUBW, SUBW)], sem).wait()
        pltpu.make_async_copy(
            table_hbm.at[next_idx_v.at[pl.ds(c * CHUNK, CHUNK)]],
            next_rows_v, nsem).wait()

        pos0 = lax.add(lane50, c * IDX_C)
        cnt = jnp.zeros((L,), jnp.float32)
        one = jnp.ones((L,), jnp.float32)
        zero = jnp.zeros((L,), jnp.float32)
        for j in range(MAX_LEN):
            v = plsc.load_gather(idx_v, [lax.add(pos0, j)])
            cnt = lax.add(cnt, lax.select(lax.ne(v, 0), one, zero))

        def row_body(r, _):
            b = r * MAX_LEN
            a0 = rows_v[b, pl.ds(0, L)]
            a1 = rows_v[b, pl.ds(L, L)]
            for j in range(1, MAX_LEN):
                a0 = lax.add(a0, rows_v[b + j, pl.ds(0, L)])
                a1 = lax.add(a1, rows_v[b + j, pl.ds(L, L)])
            p0 = lax.mul(a0, next_rows_v[r, pl.ds(0, L)])
            p1 = lax.mul(a1, next_rows_v[r, pl.ds(L, L)])
            plsc.store_scatter(prod_v, [lax.add(d16, r)], p0)
            plsc.store_scatter(prod_v, [lax.add(d16, r + L * L)], p1)
            return 0

        lax.fori_loop(0, CHUNK, row_body, 0)

        score = prod_v[pl.ds(0, L)]
        for d in range(1, EMBED_DIM):
            score = lax.add(score, prod_v[pl.ds(d * L, L)])
        out_v[pl.ds(c * CHUNK, CHUNK)] = lax.div(score, cnt)
        return 0

    lax.fori_loop(0, NCHUNK, chunk_body, 0)
    pltpu.sync_copy(out_v, out_hbm.at[pl.ds(wid * ROWS_W, ROWS_W)])


def kernel(item_seq, next_item, item_embed):
    seq_flat = item_seq.reshape(BATCH * MAX_LEN)
    mesh = plsc.VectorSubcoreMesh(core_axis_name="c", subcore_axis_name="s",
                                  num_cores=NC, num_subcores=NS)
    f = pl.kernel(
        _sbpr_kernel,
        out_type=jax.ShapeDtypeStruct((BATCH,), jnp.float32),
        mesh=mesh,
        compiler_params=pltpu.CompilerParams(needs_layout_passes=False,
                                             use_tc_tiling_on_sc=False),
        scratch_types=[
            pltpu.VMEM((IDX_W,), jnp.int32),
            pltpu.VMEM((ROWS_W,), jnp.int32),
            pltpu.VMEM((IDX_C, EMBED_DIM), jnp.float32),
            pltpu.VMEM((CHUNK, EMBED_DIM), jnp.float32),
            pltpu.VMEM((CHUNK * EMBED_DIM,), jnp.float32),
            pltpu.VMEM((ROWS_W,), jnp.float32),
            pltpu.SemaphoreType.DMA,
            pltpu.SemaphoreType.DMA,
        ],
    )
    return f(seq_flat, next_item, item_embed)


# speedup vs baseline: 3.0067x; 1.0762x over previous
import jax
import jax.numpy as jnp
from jax import lax
from jax.experimental import pallas as pl
from jax.experimental.pallas import tpu as pltpu, tpu_sc as plsc

BATCH = 16384
MAX_LEN = 50
EMBED_DIM = 32
NC, NS, L = 2, 16, 16
NW = NC * NS
ROWS_W = BATCH // NW
CHUNK = 16
NCHUNK = ROWS_W // CHUNK
IDX_W = ROWS_W * MAX_LEN
IDX_C = CHUNK * MAX_LEN


def _sbpr_kernel(seq_hbm, next_hbm, table_hbm, out_hbm,
                 idx_v, next_idx_v, rows_a, rows_b, next_a, next_b,
                 prod_v, out_v, sem_a, sem_b, nsem_a, nsem_b):
    wid = lax.axis_index("s") * NC + lax.axis_index("c")

    pltpu.sync_copy(seq_hbm.at[pl.ds(wid * IDX_W, IDX_W)], idx_v)
    pltpu.sync_copy(next_hbm.at[pl.ds(wid * ROWS_W, ROWS_W)], next_idx_v)

    iota = lax.iota(jnp.int32, L)
    lane50 = lax.mul(iota, MAX_LEN)
    d16 = lax.mul(iota, L)

    def fire(c, rows_v, next_rows_v, sem, nsem):
        pltpu.async_copy(table_hbm.at[idx_v.at[pl.ds(c * IDX_C, IDX_C)]],
                         rows_v, sem)
        pltpu.async_copy(table_hbm.at[next_idx_v.at[pl.ds(c * CHUNK, CHUNK)]],
                         next_rows_v, nsem)

    def wait(c, rows_v, next_rows_v, sem, nsem):
        pltpu.make_async_copy(table_hbm.at[idx_v.at[pl.ds(c * IDX_C, IDX_C)]],
                              rows_v, sem).wait()
        pltpu.make_async_copy(
            table_hbm.at[next_idx_v.at[pl.ds(c * CHUNK, CHUNK)]],
            next_rows_v, nsem).wait()

    def compute(c, rows_v, next_rows_v):
        pos0 = lax.add(lane50, c * IDX_C)
        cnt = jnp.zeros((L,), jnp.float32)
        one = jnp.ones((L,), jnp.float32)
        zero = jnp.zeros((L,), jnp.float32)
        for j in range(MAX_LEN):
            v = plsc.load_gather(idx_v, [lax.add(pos0, j)])
            cnt = lax.add(cnt, lax.select(lax.ne(v, 0), one, zero))

        def row_body(r, _):
            b = r * MAX_LEN
            a0 = rows_v[b, pl.ds(0, L)]
            a1 = rows_v[b, pl.ds(L, L)]
            for j in range(1, MAX_LEN):
                a0 = lax.add(a0, rows_v[b + j, pl.ds(0, L)])
                a1 = lax.add(a1, rows_v[b + j, pl.ds(L, L)])
            p0 = lax.mul(a0, next_rows_v[r, pl.ds(0, L)])
            p1 = lax.mul(a1, next_rows_v[r, pl.ds(L, L)])
            plsc.store_scatter(prod_v, [lax.add(d16, r)], p0)
            plsc.store_scatter(prod_v, [lax.add(d16, r + L * L)], p1)
            return 0

        lax.fori_loop(0, CHUNK, row_body, 0)

        score = prod_v[pl.ds(0, L)]
        for d in range(1, EMBED_DIM):
            score = lax.add(score, prod_v[pl.ds(d * L, L)])
        out_v[pl.ds(c * CHUNK, CHUNK)] = lax.div(score, cnt)

    fire(0, rows_a, next_a, sem_a, nsem_a)

    def pair_body(t, _):
        c0 = t * 2
        fire(c0 + 1, rows_b, next_b, sem_b, nsem_b)
        wait(c0, rows_a, next_a, sem_a, nsem_a)
        compute(c0, rows_a, next_a)

        @pl.when(t < NCHUNK // 2 - 1)
        def _():
            fire(c0 + 2, rows_a, next_a, sem_a, nsem_a)

        wait(c0 + 1, rows_b, next_b, sem_b, nsem_b)
        compute(c0 + 1, rows_b, next_b)
        return 0

    lax.fori_loop(0, NCHUNK // 2, pair_body, 0)
    pltpu.sync_copy(out_v, out_hbm.at[pl.ds(wid * ROWS_W, ROWS_W)])


def kernel(item_seq, next_item, item_embed):
    seq_flat = item_seq.reshape(BATCH * MAX_LEN)
    mesh = plsc.VectorSubcoreMesh(core_axis_name="c", subcore_axis_name="s",
                                  num_cores=NC, num_subcores=NS)
    f = pl.kernel(
        _sbpr_kernel,
        out_type=jax.ShapeDtypeStruct((BATCH,), jnp.float32),
        mesh=mesh,
        compiler_params=pltpu.CompilerParams(needs_layout_passes=False,
                                             use_tc_tiling_on_sc=False),
        scratch_types=[
            pltpu.VMEM((IDX_W,), jnp.int32),
            pltpu.VMEM((ROWS_W,), jnp.int32),
            pltpu.VMEM((IDX_C, EMBED_DIM), jnp.float32),
            pltpu.VMEM((IDX_C, EMBED_DIM), jnp.float32),
            pltpu.VMEM((CHUNK, EMBED_DIM), jnp.float32),
            pltpu.VMEM((CHUNK, EMBED_DIM), jnp.float32),
            pltpu.VMEM((CHUNK * EMBED_DIM,), jnp.float32),
            pltpu.VMEM((ROWS_W,), jnp.float32),
            pltpu.SemaphoreType.DMA,
            pltpu.SemaphoreType.DMA,
            pltpu.SemaphoreType.DMA,
            pltpu.SemaphoreType.DMA,
        ],
    )
    return f(seq_flat, next_item, item_embed)
